# baseline (speedup 1.0000x reference)
import functools

import jax
import jax.numpy as jnp
from jax import lax
from jax.experimental import pallas as pl
from jax.experimental.pallas import tpu as pltpu
from jax.experimental.pallas import tpu_sc as plsc

DIM = 768
NLANE = 16
NVEC = DIM // NLANE
NW = 32
KQ = 8
CH = 8
BLK = KQ * CH
EPS = 1e-5


def _lanesum(x):
    lanes = lax.iota(jnp.int32, NLANE)
    for k in (8, 4, 2, 1):
        x = x + x.at[lanes ^ k].get(mode="promise_in_bounds",
                                    unique_indices=True)
    return x


def _rsqrt16(x):
    i = plsc.bitcast(x, jnp.int32)
    y = plsc.bitcast(jnp.int32(0x5F3759DF) - (i >> 1), jnp.float32)
    for _ in range(3):
        y = y * (1.5 - 0.5 * x * y * y)
    return y


@functools.lru_cache(maxsize=None)
def _build(B, L):
    assert B % (NW * KQ) == 0 and L % CH == 0
    RPW = B // NW
    NQ = RPW // KQ
    TPW = RPW * L
    NCHUNK = L // CH
    NBLK = NCHUNK * NQ
    assert NBLK % 2 == 0 and NQ & (NQ - 1) == 0
    mesh = plsc.VectorSubcoreMesh(core_axis_name="c", subcore_axis_name="s")

    @functools.partial(
        pl.kernel,
        mesh=mesh,
        compiler_params=pltpu.CompilerParams(needs_layout_passes=False),
        out_type=jax.ShapeDtypeStruct((B * L, DIM), jnp.float32),
        scratch_types=[
            pltpu.VMEM((TPW,), jnp.int32),
            pltpu.VMEM((BLK, DIM), jnp.float32),
            pltpu.VMEM((BLK, DIM), jnp.float32),
            pltpu.VMEM((CH, DIM), jnp.float32),
            pltpu.VMEM((DIM,), jnp.float32),
            pltpu.VMEM((DIM,), jnp.float32),
            pltpu.SemaphoreType.DMA,
            pltpu.SemaphoreType.DMA,
            pltpu.SemaphoreType.DMA,
            pltpu.SemaphoreType.DMA,
        ],
    )
    def body(ids_hbm, table_hbm, bias_hbm, gamma_hbm, beta_hbm, out_hbm,
             idx_v, r0_v, r1_v, bias_v, gam_v, bet_v,
             gsem0, gsem1, osem0, osem1):
        cid = lax.axis_index("c")
        sid = lax.axis_index("s")
        wid = sid * 2 + cid
        tok0 = wid * TPW
        row0 = wid * RPW
        pltpu.sync_copy(gamma_hbm, gam_v)
        pltpu.sync_copy(beta_hbm, bet_v)
        pltpu.sync_copy(ids_hbm.at[pl.ds(tok0, TPW)], idx_v)

        def gather_copy(bi, buf, gsem):
            return pltpu.make_async_copy(
                table_hbm.at[idx_v.at[pl.ds(bi * BLK, BLK)]], buf, gsem)

        def out_copy(bi, buf, osem, k):
            lc = bi // NQ
            g = bi & (NQ - 1)
            off = (row0 + g * KQ + k) * L + lc * CH
            return pltpu.make_async_copy(
                buf.at[pl.ds(k * CH, CH)], out_hbm.at[pl.ds(off, CH)], osem)

        def compute(buf):
            def pos_loop(p, _):
                def pass1(j, carry):
                    sl = pl.ds(j * NLANE, NLANE)
                    bj = bias_v[p, sl]
                    new = []
                    for k in range(KQ):
                        x = buf[k * CH + p, sl] + bj
                        buf[k * CH + p, sl] = x
                        s, q = carry[2 * k], carry[2 * k + 1]
                        new += [s + x, q + x * x]
                    return tuple(new)

                zero = jnp.zeros((NLANE,), jnp.float32)
                accs = lax.fori_loop(0, NVEC, pass1, (zero,) * (2 * KQ),
                                     unroll=16)
                stats = []
                for k in range(KQ):
                    m = _lanesum(accs[2 * k]) * (1.0 / DIM)
                    v = _lanesum(accs[2 * k + 1]) * (1.0 / DIM) - m * m
                    r16 = _rsqrt16(v + EPS)
                    stats.append((r16, m * r16))

                def pass2(j, _):
                    sl = pl.ds(j * NLANE, NLANE)
                    gj = gam_v[sl]
                    bj = bet_v[sl]
                    for k in range(KQ):
                        x = buf[k * CH + p, sl]
                        u = x * stats[k][0] - stats[k][1]
                        buf[k * CH + p, sl] = u * gj + bj
                    return 0

                lax.fori_loop(0, NVEC, pass2, 0, unroll=16)
                return 0

            lax.fori_loop(0, CH, pos_loop, 0)

        gather_copy(0, r0_v, gsem0).start()

        bufs = (r0_v, r1_v)
        gsems = (gsem0, gsem1)
        osems = (osem0, osem1)

        def pair_loop(pp, _):
            for b in (0, 1):
                bi = pp * 2 + b
                buf, gsem = bufs[b], gsems[b]
                obuf, gsem_o, osem_o = bufs[1 - b], gsems[1 - b], osems[1 - b]

                @pl.when((bi & (NQ - 1)) == 0)
                def _():
                    pltpu.sync_copy(bias_hbm.at[pl.ds((bi // NQ) * CH, CH)],
                                    bias_v)

                gather_copy(bi, buf, gsem).wait()

                @pl.when(bi >= 1)
                def _():
                    for k in range(KQ):
                        out_copy(bi - 1, obuf, osem_o, k).wait()

                @pl.when(bi < NBLK - 1)
                def _():
                    gather_copy(bi + 1, obuf, gsem_o).start()

                compute(buf)
                for k in range(KQ):
                    out_copy(bi, buf, osems[b], k).start()
            return 0

        lax.fori_loop(0, NBLK // 2, pair_loop, 0)
        for k in range(KQ):
            out_copy(NBLK - 1, r1_v, osem1, k).wait()

    return body


def kernel(input_ids, table, pos_table, tt_table, gamma, beta):
    B, L = input_ids.shape
    bias = pos_table[:L] + tt_table[0][None, :]
    NQ = (B // NW) // KQ
    ids = (input_ids.astype(jnp.int32)
           .reshape(NW, NQ, KQ, L // CH, CH)
           .transpose(0, 3, 1, 2, 4)
           .reshape(-1))
    out = _build(B, L)(ids, table, bias, gamma, beta)
    return out.reshape(B, L, DIM)


# speedup vs baseline: 1.2922x; 1.2922x over previous
import functools

import jax
import jax.numpy as jnp
from jax import lax
from jax.experimental import pallas as pl
from jax.experimental.pallas import tpu as pltpu
from jax.experimental.pallas import tpu_sc as plsc

DIM = 768
NLANE = 16
NVEC = DIM // NLANE
NW = 32
KQ = 8
CH = 8
BLK = KQ * CH
EPS = 1e-5


def _lanesum(x):
    lanes = lax.iota(jnp.int32, NLANE)
    for k in (8, 4, 2, 1):
        x = x + x.at[lanes ^ k].get(mode="promise_in_bounds",
                                    unique_indices=True)
    return x


def _rsqrt16(x):
    i = plsc.bitcast(x, jnp.int32)
    y = plsc.bitcast(jnp.int32(0x5F3759DF) - (i >> 1), jnp.float32)
    for _ in range(2):
        y = y * (1.5 - 0.5 * x * y * y)
    return y


@functools.lru_cache(maxsize=None)
def _build(B, L):
    assert B % (NW * KQ) == 0 and L % CH == 0
    RPW = B // NW
    NQ = RPW // KQ
    TPW = RPW * L
    NCHUNK = L // CH
    NBLK = NCHUNK * NQ
    assert NBLK % 2 == 0 and NQ & (NQ - 1) == 0
    mesh = plsc.VectorSubcoreMesh(core_axis_name="c", subcore_axis_name="s")

    @functools.partial(
        pl.kernel,
        mesh=mesh,
        compiler_params=pltpu.CompilerParams(needs_layout_passes=False),
        out_type=jax.ShapeDtypeStruct((B * L, DIM), jnp.float32),
        scratch_types=[
            pltpu.VMEM((TPW,), jnp.int32),
            pltpu.VMEM((BLK, DIM), jnp.float32),
            pltpu.VMEM((BLK, DIM), jnp.float32),
            pltpu.VMEM((CH, DIM), jnp.float32),
            pltpu.VMEM((DIM,), jnp.float32),
            pltpu.VMEM((DIM,), jnp.float32),
            pltpu.SemaphoreType.DMA,
            pltpu.SemaphoreType.DMA,
            pltpu.SemaphoreType.DMA,
            pltpu.SemaphoreType.DMA,
        ],
    )
    def body(ids_hbm, table_hbm, bias_hbm, gamma_hbm, beta_hbm, out_hbm,
             idx_v, r0_v, r1_v, bias_v, gam_v, bet_v,
             gsem0, gsem1, osem0, osem1):
        cid = lax.axis_index("c")
        sid = lax.axis_index("s")
        wid = sid * 2 + cid
        tok0 = wid * TPW
        row0 = wid * RPW
        pltpu.sync_copy(gamma_hbm, gam_v)
        pltpu.sync_copy(beta_hbm, bet_v)
        pltpu.sync_copy(ids_hbm.at[pl.ds(tok0, TPW)], idx_v)

        def gather_copy(bi, buf, gsem):
            return pltpu.make_async_copy(
                table_hbm.at[idx_v.at[pl.ds(bi * BLK, BLK)]], buf, gsem)

        def out_copy(bi, buf, osem, k):
            lc = bi // NQ
            g = bi & (NQ - 1)
            off = (row0 + g * KQ + k) * L + lc * CH
            return pltpu.make_async_copy(
                buf.at[pl.ds(k * CH, CH)], out_hbm.at[pl.ds(off, CH)], osem)

        def compute(buf):
            def pos_loop(p, _):
                def pass1(j, carry):
                    sl = pl.ds(j * NLANE, NLANE)
                    bj = bias_v[p, sl]
                    new = []
                    for k in range(KQ):
                        x = buf[k * CH + p, sl] + bj
                        buf[k * CH + p, sl] = x
                        s, q = carry[2 * k], carry[2 * k + 1]
                        new += [s + x, q + x * x]
                    return tuple(new)

                zero = jnp.zeros((NLANE,), jnp.float32)
                accs = lax.fori_loop(0, NVEC, pass1, (zero,) * (2 * KQ),
                                     unroll=8)
                stats = []
                for k in range(KQ):
                    m = _lanesum(accs[2 * k]) * (1.0 / DIM)
                    v = _lanesum(accs[2 * k + 1]) * (1.0 / DIM) - m * m
                    r16 = _rsqrt16(v + EPS)
                    stats.append((r16, m * r16))

                def pass2(j, _):
                    sl = pl.ds(j * NLANE, NLANE)
                    gj = gam_v[sl]
                    bj = bet_v[sl]
                    for k in range(KQ):
                        x = buf[k * CH + p, sl]
                        u = x * stats[k][0] - stats[k][1]
                        buf[k * CH + p, sl] = u * gj + bj
                    return 0

                lax.fori_loop(0, NVEC, pass2, 0, unroll=8)
                return 0

            lax.fori_loop(0, CH, pos_loop, 0)

        gather_copy(0, r0_v, gsem0).start()

        bufs = (r0_v, r1_v)
        gsems = (gsem0, gsem1)
        osems = (osem0, osem1)

        def pair_loop(pp, _):
            for b in (0, 1):
                bi = pp * 2 + b
                buf, gsem = bufs[b], gsems[b]
                obuf, gsem_o, osem_o = bufs[1 - b], gsems[1 - b], osems[1 - b]

                @pl.when((bi & (NQ - 1)) == 0)
                def _():
                    pltpu.sync_copy(bias_hbm.at[pl.ds((bi // NQ) * CH, CH)],
                                    bias_v)

                gather_copy(bi, buf, gsem).wait()

                @pl.when(bi >= 1)
                def _():
                    for k in range(KQ):
                        out_copy(bi - 1, obuf, osem_o, k).wait()

                @pl.when(bi < NBLK - 1)
                def _():
                    gather_copy(bi + 1, obuf, gsem_o).start()

                compute(buf)
                for k in range(KQ):
                    out_copy(bi, buf, osems[b], k).start()
            return 0

        lax.fori_loop(0, NBLK // 2, pair_loop, 0)
        for k in range(KQ):
            out_copy(NBLK - 1, r1_v, osem1, k).wait()

    return body


def kernel(input_ids, table, pos_table, tt_table, gamma, beta):
    B, L = input_ids.shape
    bias = pos_table[:L] + tt_table[0][None, :]
    NQ = (B // NW) // KQ
    ids = (input_ids.astype(jnp.int32)
           .reshape(NW, NQ, KQ, L // CH, CH)
           .transpose(0, 3, 1, 2, 4)
           .reshape(-1))
    out = _build(B, L)(ids, table, bias, gamma, beta)
    return out.reshape(B, L, DIM)


# speedup vs baseline: 1.6777x; 1.2984x over previous
import functools

import jax
import jax.numpy as jnp
from jax import lax
from jax.experimental import pallas as pl
from jax.experimental.pallas import tpu as pltpu
from jax.experimental.pallas import tpu_sc as plsc

DIM = 768
NLANE = 16
NVEC = DIM // NLANE
NW = 32
KQ = 8
CH = 8
BLK = KQ * CH
EPS = 1e-5


def _lanesum(x):
    lanes = lax.iota(jnp.int32, NLANE)
    for k in (8, 4, 2, 1):
        x = x + x.at[lanes ^ k].get(mode="promise_in_bounds",
                                    unique_indices=True)
    return x


def _rsqrt16(x):
    i = plsc.bitcast(x, jnp.int32)
    y = plsc.bitcast(jnp.int32(0x5F3759DF) - (i >> 1), jnp.float32)
    for _ in range(2):
        y = y * (1.5 - 0.5 * x * y * y)
    return y


@functools.lru_cache(maxsize=None)
def _build(B, L, with_gb):
    assert B % (NW * KQ) == 0 and L % CH == 0
    RPW = B // NW
    NQ = RPW // KQ
    TPW = RPW * L
    NCHUNK = L // CH
    NBLK = NCHUNK * NQ
    assert NBLK % 2 == 0 and NQ & (NQ - 1) == 0
    mesh = plsc.VectorSubcoreMesh(core_axis_name="c", subcore_axis_name="s")

    scratch = [
        pltpu.VMEM((TPW,), jnp.int32),
        pltpu.VMEM((BLK, DIM), jnp.float32),
        pltpu.VMEM((BLK, DIM), jnp.float32),
        pltpu.VMEM((CH, DIM), jnp.float32),
    ]
    if with_gb:
        scratch += [
            pltpu.VMEM((DIM,), jnp.float32),
            pltpu.VMEM((DIM,), jnp.float32),
        ]
    scratch += [
        pltpu.SemaphoreType.DMA,
        pltpu.SemaphoreType.DMA,
        pltpu.SemaphoreType.DMA,
        pltpu.SemaphoreType.DMA,
    ]

    @functools.partial(
        pl.kernel,
        mesh=mesh,
        compiler_params=pltpu.CompilerParams(needs_layout_passes=False),
        out_type=jax.ShapeDtypeStruct((B * L, DIM), jnp.float32),
        scratch_types=scratch,
    )
    def body(*args):
        if with_gb:
            (ids_hbm, table_hbm, bias_hbm, gamma_hbm, beta_hbm, out_hbm,
             idx_v, r0_v, r1_v, bias_v, gam_v, bet_v,
             gsem0, gsem1, osem0, osem1) = args
        else:
            (ids_hbm, table_hbm, bias_hbm, out_hbm,
             idx_v, r0_v, r1_v, bias_v,
             gsem0, gsem1, osem0, osem1) = args
        cid = lax.axis_index("c")
        sid = lax.axis_index("s")
        wid = sid * 2 + cid
        tok0 = wid * TPW
        row0 = wid * RPW
        if with_gb:
            pltpu.sync_copy(gamma_hbm, gam_v)
            pltpu.sync_copy(beta_hbm, bet_v)
        pltpu.sync_copy(ids_hbm.at[pl.ds(tok0, TPW)], idx_v)

        def gather_copy(bi, buf, gsem):
            return pltpu.make_async_copy(
                table_hbm.at[idx_v.at[pl.ds(bi * BLK, BLK)]], buf, gsem)

        def out_copy(bi, buf, osem, k):
            lc = bi // NQ
            g = bi & (NQ - 1)
            off = (row0 + g * KQ + k) * L + lc * CH
            return pltpu.make_async_copy(
                buf.at[pl.ds(k * CH, CH)], out_hbm.at[pl.ds(off, CH)], osem)

        def compute(buf):
            def pos_loop(p, _):
                def pass1(j, carry):
                    sl = pl.ds(j * NLANE, NLANE)
                    bj = bias_v[p, sl]
                    new = []
                    for k in range(KQ):
                        x = buf[k * CH + p, sl] + bj
                        buf[k * CH + p, sl] = x
                        s, q = carry[2 * k], carry[2 * k + 1]
                        new += [s + x, q + x * x]
                    return tuple(new)

                zero = jnp.zeros((NLANE,), jnp.float32)
                accs = lax.fori_loop(0, NVEC, pass1, (zero,) * (2 * KQ),
                                     unroll=8)
                stats = []
                for k in range(KQ):
                    m = _lanesum(accs[2 * k]) * (1.0 / DIM)
                    v = _lanesum(accs[2 * k + 1]) * (1.0 / DIM) - m * m
                    r16 = _rsqrt16(v + EPS)
                    stats.append((r16, m * r16))

                def pass2(j, _):
                    sl = pl.ds(j * NLANE, NLANE)
                    if with_gb:
                        gj = gam_v[sl]
                        bj = bet_v[sl]
                    for k in range(KQ):
                        x = buf[k * CH + p, sl]
                        u = x * stats[k][0] - stats[k][1]
                        if with_gb:
                            u = u * gj + bj
                        buf[k * CH + p, sl] = u
                    return 0

                lax.fori_loop(0, NVEC, pass2, 0, unroll=8)
                return 0

            lax.fori_loop(0, CH, pos_loop, 0)

        gather_copy(0, r0_v, gsem0).start()

        bufs = (r0_v, r1_v)
        gsems = (gsem0, gsem1)
        osems = (osem0, osem1)

        def pair_loop(pp, _):
            for b in (0, 1):
                bi = pp * 2 + b
                buf, gsem = bufs[b], gsems[b]
                obuf, gsem_o, osem_o = bufs[1 - b], gsems[1 - b], osems[1 - b]

                @pl.when((bi & (NQ - 1)) == 0)
                def _():
                    pltpu.sync_copy(bias_hbm.at[pl.ds((bi // NQ) * CH, CH)],
                                    bias_v)

                gather_copy(bi, buf, gsem).wait()

                @pl.when(bi >= 1)
                def _():
                    for k in range(KQ):
                        out_copy(bi - 1, obuf, osem_o, k).wait()

                @pl.when(bi < NBLK - 1)
                def _():
                    gather_copy(bi + 1, obuf, gsem_o).start()

                compute(buf)
                for k in range(KQ):
                    out_copy(bi, buf, osems[b], k).start()
            return 0

        lax.fori_loop(0, NBLK // 2, pair_loop, 0)
        for k in range(KQ):
            out_copy(NBLK - 1, r1_v, osem1, k).wait()

    return body


def kernel(input_ids, table, pos_table, tt_table, gamma, beta):
    B, L = input_ids.shape
    bias = pos_table[:L] + tt_table[0][None, :]
    NQ = (B // NW) // KQ
    ids = (input_ids.astype(jnp.int32)
           .reshape(NW, NQ, KQ, L // CH, CH)
           .transpose(0, 3, 1, 2, 4)
           .reshape(-1))
    identity_affine = jnp.logical_and(jnp.all(gamma == 1.0),
                                      jnp.all(beta == 0.0))
    out = lax.cond(
        identity_affine,
        lambda: _build(B, L, False)(ids, table, bias),
        lambda: _build(B, L, True)(ids, table, bias, gamma, beta),
    )
    return out.reshape(B, L, DIM)
